# baseline (speedup 1.0000x reference)
import functools

import jax
import jax.numpy as jnp
from jax import lax
from jax.experimental import pallas as pl
from jax.experimental.pallas import tpu as pltpu
from jax.experimental.pallas import tpu_sc as plsc

N_RAYS = 65536
N_W = 254
N_BINS = 255
N_S = 256
N_OP = 257
L = 16

NC = 2
NSUB = 16
NWORK = NC * NSUB
ROWS_PER_W = N_RAYS // NWORK
G = 16
NGRP = ROWS_PER_W // G

W_WORDS = G * N_W
B_WORDS = G * N_BINS
O_WORDS = G * N_OP
H_WORDS = 257 * L


def _sc_body(wf, bf, out, w0, w1, b0, b1, o0, o1, cdf0, cdf1, cdfn0, cdfn1,
             hist0, hist1, sw0, sw1, sb0, sb1, so0, so1):
    cid = lax.axis_index("c")
    sid = lax.axis_index("s")
    wid = sid * NC + cid
    row0 = wid * ROWS_PER_W

    wbufs = (w0, w1)
    bbufs = (b0, b1)
    obufs = (o0, o1)
    cdfbs = (cdf0, cdf1)
    cdfnbs = (cdfn0, cdfn1)
    histbs = (hist0, hist1)
    swse = (sw0, sw1)
    sbse = (sb0, sb1)
    sose = (so0, so1)

    def in_copies(grp, slot):
        base = row0 + grp * G
        cw = pltpu.make_async_copy(
            wf.at[pl.ds(base * N_W, W_WORDS)], wbufs[slot], swse[slot])
        cb = pltpu.make_async_copy(
            bf.at[pl.ds(base * N_BINS, B_WORDS)], bbufs[slot], sbse[slot])
        return cw, cb

    def out_copy(grp, slot):
        base = row0 + grp * G
        return pltpu.make_async_copy(
            obufs[slot].at[:, pl.ds(0, N_S)],
            out.at[pl.ds(base, G)], sose[slot])

    lane = lax.iota(jnp.int32, L)
    lane254 = lane * N_W
    lane255 = lane * N_BINS
    lane257 = lane * N_OP
    ones_i = jnp.ones((L,), jnp.int32)
    zeros_i = jnp.zeros((L,), jnp.int32)
    u_step = jnp.float32(1.0 / 256.0)
    u_init = jnp.full((L,), 0.5 / 256.0, jnp.float32)

    def hclear(i, acc):
        hist0[pl.ds(i * L, L)] = zeros_i
        hist1[pl.ds(i * L, L)] = zeros_i
        return acc

    lax.fori_loop(0, 257, hclear, jnp.int32(0))

    def compute_group(slot):
        wbuf = wbufs[slot]
        bbuf = bbufs[slot]
        obuf = obufs[slot]
        cdfb = cdfbs[slot]
        cdfnb = cdfnbs[slot]
        histb = histbs[slot]

        @plsc.parallel_loop(0, N_W, unroll=8,
                            carry=jnp.zeros((L,), jnp.float32))
        def p1(k, run):
            cdfb[pl.ds(k * L, L)] = run
            v = plsc.load_gather(wbuf, [lane254 + k])
            return run + (v + 1e-5)

        run = p1
        cdfb[pl.ds(N_W * L, L)] = run
        inv = 1.0 / run
        plsc.subcore_barrier()

        @plsc.parallel_loop(0, N_BINS, unroll=8)
        def p2(k):
            e = cdfb[pl.ds(k * L, L)]
            en = e * inv
            cdfnb[pl.ds(k * L, L)] = en
            y = en * 256.0 - 0.5
            y0 = jnp.maximum(y, 0.0)
            ti = y0.astype(jnp.int32)
            st = ti + jnp.where(ti.astype(jnp.float32) < y0, 1, 0)
            st = jnp.minimum(st, 256)
            plsc.addupdate_scatter(histb, [(st << 4) + lane], ones_i)

        plsc.subcore_barrier()

        @plsc.parallel_loop(0, N_S, unroll=4, carry=(zeros_i, u_init))
        def p3(j, carry):
            inds, u = carry
            h = histb[pl.ds(j * L, L)]
            inds = inds + h
            below = jnp.maximum(inds - 1, 0)
            above = jnp.minimum(inds, N_BINS - 1)
            c0 = plsc.load_gather(cdfnb, [(below << 4) + lane])
            c1 = plsc.load_gather(cdfnb, [(above << 4) + lane])
            g0 = plsc.load_gather(bbuf, [lane255 + below])
            g1 = plsc.load_gather(bbuf, [lane255 + above])
            den = c1 - c0
            den = jnp.where(den < 1e-5, 1.0, den)
            t = (u - c0) / den
            plsc.store_scatter(obuf, [lane, jnp.full((L,), j, jnp.int32)],
                               g0 + t * (g1 - g0))
            return inds, u + u_step

        plsc.subcore_barrier()

        @plsc.parallel_loop(0, 257, unroll=8)
        def hcl(i):
            histb[pl.ds(i * L, L)] = zeros_i

    for s in range(2):
        cw, cb = in_copies(s, s)
        cw.start()
        cb.start()

    def step(g, acc):
        for slot in range(2):
            i = 2 * g + slot
            cw, cb = in_copies(i, slot)
            cw.wait()
            cb.wait()

            @pl.when(g > 0)
            def _():
                out_copy(i - 2, slot).wait()

            compute_group(slot)
            out_copy(i, slot).start()

            @pl.when(i + 2 < NGRP)
            def _():
                cw2, cb2 = in_copies(i + 2, slot)
                cw2.start()
                cb2.start()
        return acc

    lax.fori_loop(0, NGRP // 2, step, jnp.int32(0))
    out_copy(NGRP - 2, 0).wait()
    out_copy(NGRP - 1, 1).wait()


def _sample_pdf(bins, weights):
    assert bins.shape == (N_RAYS, N_BINS)
    wf = weights.reshape(-1)
    bf = bins.reshape(-1)

    mesh = plsc.VectorSubcoreMesh(core_axis_name="c", subcore_axis_name="s")
    run = functools.partial(
        pl.kernel,
        out_type=jax.ShapeDtypeStruct((N_RAYS, N_S), jnp.float32),
        mesh=mesh,
        compiler_params=pltpu.CompilerParams(needs_layout_passes=False),
        scratch_types=[
            pltpu.VMEM((W_WORDS,), jnp.float32),
            pltpu.VMEM((W_WORDS,), jnp.float32),
            pltpu.VMEM((B_WORDS,), jnp.float32),
            pltpu.VMEM((B_WORDS,), jnp.float32),
            pltpu.VMEM((G, N_OP), jnp.float32),
            pltpu.VMEM((G, N_OP), jnp.float32),
            pltpu.VMEM((N_BINS * L + L,), jnp.float32),
            pltpu.VMEM((N_BINS * L + L,), jnp.float32),
            pltpu.VMEM((N_BINS * L + L,), jnp.float32),
            pltpu.VMEM((N_BINS * L + L,), jnp.float32),
            pltpu.VMEM((H_WORDS,), jnp.int32),
            pltpu.VMEM((H_WORDS,), jnp.int32),
            pltpu.SemaphoreType.DMA,
            pltpu.SemaphoreType.DMA,
            pltpu.SemaphoreType.DMA,
            pltpu.SemaphoreType.DMA,
            pltpu.SemaphoreType.DMA,
            pltpu.SemaphoreType.DMA,
        ],
    )(_sc_body)
    return run(wf, bf)


def kernel(bins, weights, n_samples):
    del n_samples
    return _sample_pdf(bins, weights)


# speedup vs baseline: 1.1278x; 1.1278x over previous
import functools

import jax
import jax.numpy as jnp
from jax import lax
from jax.experimental import pallas as pl
from jax.experimental.pallas import tpu as pltpu
from jax.experimental.pallas import tpu_sc as plsc

N_RAYS = 65536
N_W = 254
N_BINS = 255
N_S = 256
N_OP = 257
L = 16

NC = 2
NSUB = 16
NWORK = NC * NSUB
ROWS_PER_W = N_RAYS // NWORK
G = 16
NGRP = ROWS_PER_W // G

W_WORDS = G * N_W
B_WORDS = G * N_BINS
O_WORDS = G * N_OP
H_WORDS = 257 * L


def _sc_body(wf, bf, out, w0, w1, b0, b1, o0, o1, cdf0, cdf1, cdfn0, cdfn1,
             hist0, hist1, sw0, sw1, sb0, sb1, so0, so1):
    cid = lax.axis_index("c")
    sid = lax.axis_index("s")
    wid = sid * NC + cid
    row0 = wid * ROWS_PER_W

    wbufs = (w0, w1)
    bbufs = (b0, b1)
    obufs = (o0, o1)
    cdfbs = (cdf0, cdf1)
    cdfnbs = (cdfn0, cdfn1)
    histbs = (hist0, hist1)
    swse = (sw0, sw1)
    sbse = (sb0, sb1)
    sose = (so0, so1)

    def in_copies(grp, slot):
        base = row0 + grp * G
        cw = pltpu.make_async_copy(
            wf.at[pl.ds(base * N_W, W_WORDS)], wbufs[slot], swse[slot])
        cb = pltpu.make_async_copy(
            bf.at[pl.ds(base * N_BINS, B_WORDS)], bbufs[slot], sbse[slot])
        return cw, cb

    def out_copy(grp, slot):
        base = row0 + grp * G
        return pltpu.make_async_copy(
            obufs[slot], out.at[pl.ds(base * N_OP, O_WORDS)], sose[slot])

    lane = lax.iota(jnp.int32, L)
    lane254 = lane * N_W
    lane255 = lane * N_BINS
    lane257 = lane * N_OP
    ones_i = jnp.ones((L,), jnp.int32)
    zeros_i = jnp.zeros((L,), jnp.int32)
    u_step = jnp.float32(1.0 / 256.0)
    u_init = jnp.full((L,), 0.5 / 256.0, jnp.float32)

    def hclear(i, acc):
        hist0[pl.ds(i * L, L)] = zeros_i
        hist1[pl.ds(i * L, L)] = zeros_i
        return acc

    lax.fori_loop(0, 257, hclear, jnp.int32(0))

    def compute_group(slot):
        wbuf = wbufs[slot]
        bbuf = bbufs[slot]
        obuf = obufs[slot]
        cdfb = cdfbs[slot]
        cdfnb = cdfnbs[slot]
        histb = histbs[slot]

        @plsc.parallel_loop(0, N_W, unroll=8,
                            carry=jnp.zeros((L,), jnp.float32))
        def p1(k, run):
            cdfb[pl.ds(k * L, L)] = run
            v = plsc.load_gather(wbuf, [lane254 + k])
            return run + (v + 1e-5)

        run = p1
        cdfb[pl.ds(N_W * L, L)] = run
        inv = 1.0 / run

        @plsc.parallel_loop(0, N_BINS, unroll=8)
        def p2(k):
            e = cdfb[pl.ds(k * L, L)]
            en = e * inv
            cdfnb[pl.ds(k * L, L)] = en
            y = en * 256.0 - 0.5
            y0 = jnp.maximum(y, 0.0)
            ti = y0.astype(jnp.int32)
            st = ti + jnp.where(ti.astype(jnp.float32) < y0, 1, 0)
            st = jnp.minimum(st, 256)
            plsc.addupdate_scatter(histb, [(st << 4) + lane], ones_i)


        @plsc.parallel_loop(0, N_S, unroll=4, carry=(zeros_i, u_init))
        def p3(j, carry):
            inds, u = carry
            h = histb[pl.ds(j * L, L)]
            inds = inds + h
            below = jnp.maximum(inds - 1, 0)
            above = jnp.minimum(inds, N_BINS - 1)
            c0 = plsc.load_gather(cdfnb, [(below << 4) + lane])
            c1 = plsc.load_gather(cdfnb, [(above << 4) + lane])
            g0 = plsc.load_gather(bbuf, [lane255 + below])
            g1 = plsc.load_gather(bbuf, [lane255 + above])
            den = c1 - c0
            den = jnp.where(den < 1e-5, 1.0, den)
            t = (u - c0) / den
            plsc.store_scatter(obuf, [lane257 + j], g0 + t * (g1 - g0))
            return inds, u + u_step

        plsc.subcore_barrier()

        @plsc.parallel_loop(0, 257, unroll=8)
        def hcl(i):
            histb[pl.ds(i * L, L)] = zeros_i

    for s in range(2):
        cw, cb = in_copies(s, s)
        cw.start()
        cb.start()

    def step(g, acc):
        for slot in range(2):
            i = 2 * g + slot
            cw, cb = in_copies(i, slot)
            cw.wait()
            cb.wait()

            @pl.when(g > 0)
            def _():
                out_copy(i - 2, slot).wait()

            compute_group(slot)
            out_copy(i, slot).start()

            @pl.when(i + 2 < NGRP)
            def _():
                cw2, cb2 = in_copies(i + 2, slot)
                cw2.start()
                cb2.start()
        return acc

    lax.fori_loop(0, NGRP // 2, step, jnp.int32(0))
    out_copy(NGRP - 2, 0).wait()
    out_copy(NGRP - 1, 1).wait()


def _sample_pdf(bins, weights):
    assert bins.shape == (N_RAYS, N_BINS)
    wf = weights.reshape(-1)
    bf = bins.reshape(-1)

    mesh = plsc.VectorSubcoreMesh(core_axis_name="c", subcore_axis_name="s")
    run = functools.partial(
        pl.kernel,
        out_type=jax.ShapeDtypeStruct((N_RAYS * N_OP,), jnp.float32),
        mesh=mesh,
        compiler_params=pltpu.CompilerParams(needs_layout_passes=False),
        scratch_types=[
            pltpu.VMEM((W_WORDS,), jnp.float32),
            pltpu.VMEM((W_WORDS,), jnp.float32),
            pltpu.VMEM((B_WORDS,), jnp.float32),
            pltpu.VMEM((B_WORDS,), jnp.float32),
            pltpu.VMEM((O_WORDS,), jnp.float32),
            pltpu.VMEM((O_WORDS,), jnp.float32),
            pltpu.VMEM((N_BINS * L + L,), jnp.float32),
            pltpu.VMEM((N_BINS * L + L,), jnp.float32),
            pltpu.VMEM((N_BINS * L + L,), jnp.float32),
            pltpu.VMEM((N_BINS * L + L,), jnp.float32),
            pltpu.VMEM((H_WORDS,), jnp.int32),
            pltpu.VMEM((H_WORDS,), jnp.int32),
            pltpu.SemaphoreType.DMA,
            pltpu.SemaphoreType.DMA,
            pltpu.SemaphoreType.DMA,
            pltpu.SemaphoreType.DMA,
            pltpu.SemaphoreType.DMA,
            pltpu.SemaphoreType.DMA,
        ],
    )(_sc_body)
    outp = run(wf, bf)
    return outp.reshape(N_RAYS, N_OP)[:, :N_S]


def kernel(bins, weights, n_samples):
    del n_samples
    return _sample_pdf(bins, weights)


# speedup vs baseline: 1.1279x; 1.0001x over previous
import functools

import jax
import jax.numpy as jnp
from jax import lax
from jax.experimental import pallas as pl
from jax.experimental.pallas import tpu as pltpu
from jax.experimental.pallas import tpu_sc as plsc

N_RAYS = 65536
N_W = 254
N_BINS = 255
N_S = 256
N_OP = 257
L = 16

NC = 2
NSUB = 16
NWORK = NC * NSUB
ROWS_PER_W = N_RAYS // NWORK
G = 16
NGRP = ROWS_PER_W // G

W_WORDS = G * N_W
B_WORDS = G * N_BINS
O_WORDS = G * N_OP
H_WORDS = 257 * L


def _sc_body(wf, bf, out, w0, w1, b0, b1, o0, o1, cdf0, cdf1, cdfn0, cdfn1,
             hist0, hist1, sw0, sw1, sb0, sb1, so0, so1):
    cid = lax.axis_index("c")
    sid = lax.axis_index("s")
    wid = sid * NC + cid
    row0 = wid * ROWS_PER_W

    wbufs = (w0, w1)
    bbufs = (b0, b1)
    obufs = (o0, o1)
    cdfbs = (cdf0, cdf1)
    cdfnbs = (cdfn0, cdfn1)
    histbs = (hist0, hist1)
    swse = (sw0, sw1)
    sbse = (sb0, sb1)
    sose = (so0, so1)

    def in_copies(grp, slot):
        base = row0 + grp * G
        cw = pltpu.make_async_copy(
            wf.at[pl.ds(base * N_W, W_WORDS)], wbufs[slot], swse[slot])
        cb = pltpu.make_async_copy(
            bf.at[pl.ds(base * N_BINS, B_WORDS)], bbufs[slot], sbse[slot])
        return cw, cb

    def out_copy(grp, slot):
        base = row0 + grp * G
        return pltpu.make_async_copy(
            obufs[slot], out.at[pl.ds(base * N_OP, O_WORDS)], sose[slot])

    lane = lax.iota(jnp.int32, L)
    lane254 = lane * N_W
    lane255 = lane * N_BINS
    lane257 = lane * N_OP
    ones_i = jnp.ones((L,), jnp.int32)
    zeros_i = jnp.zeros((L,), jnp.int32)
    u_step = jnp.float32(1.0 / 256.0)
    u_init = jnp.full((L,), 0.5 / 256.0, jnp.float32)

    def hclear(i, acc):
        hist0[pl.ds(i * L, L)] = zeros_i
        hist1[pl.ds(i * L, L)] = zeros_i
        return acc

    lax.fori_loop(0, 257, hclear, jnp.int32(0))

    def compute_group(slot):
        wbuf = wbufs[slot]
        bbuf = bbufs[slot]
        obuf = obufs[slot]
        cdfb = cdfbs[slot]
        cdfnb = cdfnbs[slot]
        histb = histbs[slot]

        @plsc.parallel_loop(0, N_W, unroll=8,
                            carry=jnp.zeros((L,), jnp.float32))
        def p1(k, run):
            cdfb[pl.ds(k * L, L)] = run
            v = plsc.load_gather(wbuf, [lane254 + k])
            return run + (v + 1e-5)

        run = p1
        cdfb[pl.ds(N_W * L, L)] = run
        inv = 1.0 / run

        @plsc.parallel_loop(0, N_BINS, unroll=8)
        def p2(k):
            e = cdfb[pl.ds(k * L, L)]
            en = e * inv
            cdfnb[pl.ds(k * L, L)] = en
            y = en * 256.0 - 0.5
            y0 = jnp.maximum(y, 0.0)
            ti = y0.astype(jnp.int32)
            st = ti + jnp.where(ti.astype(jnp.float32) < y0, 1, 0)
            st = jnp.minimum(st, 256)
            plsc.addupdate_scatter(histb, [(st << 4) + lane], ones_i)


        @plsc.parallel_loop(0, N_S, unroll=4, carry=(zeros_i, u_init))
        def p3(j, carry):
            inds, u = carry
            h = histb[pl.ds(j * L, L)]
            inds = inds + h
            below = jnp.maximum(inds - 1, 0)
            above = jnp.minimum(inds, N_BINS - 1)
            c0 = plsc.load_gather(cdfnb, [(below << 4) + lane])
            c1 = plsc.load_gather(cdfnb, [(above << 4) + lane])
            g0 = plsc.load_gather(bbuf, [lane255 + below])
            g1 = plsc.load_gather(bbuf, [lane255 + above])
            den = c1 - c0
            den = jnp.where(den < 1e-5, 1.0, den)
            t = (u - c0) / den
            plsc.store_scatter(obuf, [lane257 + j], g0 + t * (g1 - g0))
            return inds, u + u_step

        plsc.subcore_barrier()

    for s in range(2):
        cw, cb = in_copies(s, s)
        cw.start()
        cb.start()

    def step(g, acc):
        for slot in range(2):
            i = 2 * g + slot
            cw, cb = in_copies(i, slot)
            cw.wait()
            cb.wait()

            @pl.when(g > 0)
            def _():
                out_copy(i - 2, slot).wait()

            compute_group(slot)
            out_copy(i, slot).start()

            histb = histbs[slot]

            @plsc.parallel_loop(0, 257, unroll=8)
            def hcl(hi):
                histb[pl.ds(hi * L, L)] = zeros_i

            @pl.when(i + 2 < NGRP)
            def _():
                cw2, cb2 = in_copies(i + 2, slot)
                cw2.start()
                cb2.start()
        return acc

    lax.fori_loop(0, NGRP // 2, step, jnp.int32(0))
    out_copy(NGRP - 2, 0).wait()
    out_copy(NGRP - 1, 1).wait()


def _sample_pdf(bins, weights):
    assert bins.shape == (N_RAYS, N_BINS)
    wf = weights.reshape(-1)
    bf = bins.reshape(-1)

    mesh = plsc.VectorSubcoreMesh(core_axis_name="c", subcore_axis_name="s")
    run = functools.partial(
        pl.kernel,
        out_type=jax.ShapeDtypeStruct((N_RAYS * N_OP,), jnp.float32),
        mesh=mesh,
        compiler_params=pltpu.CompilerParams(needs_layout_passes=False),
        scratch_types=[
            pltpu.VMEM((W_WORDS,), jnp.float32),
            pltpu.VMEM((W_WORDS,), jnp.float32),
            pltpu.VMEM((B_WORDS,), jnp.float32),
            pltpu.VMEM((B_WORDS,), jnp.float32),
            pltpu.VMEM((O_WORDS,), jnp.float32),
            pltpu.VMEM((O_WORDS,), jnp.float32),
            pltpu.VMEM((N_BINS * L + L,), jnp.float32),
            pltpu.VMEM((N_BINS * L + L,), jnp.float32),
            pltpu.VMEM((N_BINS * L + L,), jnp.float32),
            pltpu.VMEM((N_BINS * L + L,), jnp.float32),
            pltpu.VMEM((H_WORDS,), jnp.int32),
            pltpu.VMEM((H_WORDS,), jnp.int32),
            pltpu.SemaphoreType.DMA,
            pltpu.SemaphoreType.DMA,
            pltpu.SemaphoreType.DMA,
            pltpu.SemaphoreType.DMA,
            pltpu.SemaphoreType.DMA,
            pltpu.SemaphoreType.DMA,
        ],
    )(_sc_body)
    outp = run(wf, bf)
    return outp.reshape(N_RAYS, N_OP)[:, :N_S]


def kernel(bins, weights, n_samples):
    del n_samples
    return _sample_pdf(bins, weights)
